# Initial kernel scaffold; baseline (speedup 1.0000x reference)
import jax
import jax.numpy as jnp
from jax.experimental import pallas as pl


def kernel(positions, angles, grid_values):
    raise NotImplementedError("write your pallas kernel here")


# speedup vs baseline: 9.6298x; 9.6298x over previous
import functools
import math

import jax
import jax.numpy as jnp
from jax import lax
from jax.experimental import pallas as pl
from jax.experimental.pallas import tpu as pltpu
from jax.experimental.pallas import tpu_sc as plsc

N = 50000
F = 64
A = 8
H = 100
W = 100
NC = 2
NS = 16
NW = NC * NS
L = 16
PTS_PER_W = 1568
NP = NW * PTS_PER_W
GROUPS = PTS_PER_W // L
OMEGA = 2.0 * math.pi / A
SIN_OMEGA = math.sin(OMEGA)


def _sinpoly(t):
    t2 = t * t
    return t * (1.0 + t2 * (-1.0 / 6.0 + t2 * (1.0 / 120.0 - t2 * (1.0 / 5040.0))))


@functools.partial(
    pl.kernel,
    out_type=jax.ShapeDtypeStruct((NP * F,), jnp.float32),
    mesh=plsc.VectorSubcoreMesh(
        core_axis_name="c", subcore_axis_name="s", num_cores=NC, num_subcores=NS
    ),
    scratch_types=[
        pltpu.VMEM((PTS_PER_W,), jnp.float32),
        pltpu.VMEM((PTS_PER_W,), jnp.float32),
        pltpu.VMEM((PTS_PER_W,), jnp.float32),
        pltpu.VMEM((16,), jnp.float32),
        pltpu.VMEM((4 * L,), jnp.int32),
        pltpu.VMEM((4 * L, 2 * F), jnp.float32),
        pltpu.VMEM((PTS_PER_W * F,), jnp.float32),
        pltpu.SemaphoreType.DMA,
    ],
    compiler_params=pltpu.CompilerParams(needs_layout_passes=False),
)
def _interp_sc(table, xs, ys, angs, az, out_hbm,
               x_v, y_v, a_v, az_v, idx_v, rows_v, out_v, sem):
    wid = lax.axis_index("s") * NC + lax.axis_index("c")
    base = pl.multiple_of(wid * PTS_PER_W, 8)
    pltpu.sync_copy(xs.at[pl.ds(base, PTS_PER_W)], x_v)
    pltpu.sync_copy(ys.at[pl.ds(base, PTS_PER_W)], y_v)
    pltpu.sync_copy(angs.at[pl.ds(base, PTS_PER_W)], a_v)
    pltpu.sync_copy(az, az_v)

    iota = lax.iota(jnp.int32, L)
    row_ids = [iota + k * L for k in range(4)]

    def dim_interp(v, n):
        cv = v.astype(jnp.int32)
        ceil = jnp.where(v > cv.astype(jnp.float32), cv + 1, cv)
        r = jnp.minimum(ceil, n - 1)
        lft = jnp.maximum(r - 1, 0)
        dl = jnp.maximum(v - lft.astype(jnp.float32), 0.0)
        dr = jnp.maximum(r.astype(jnp.float32) - v, 0.0)
        b0 = (dl == 0.0) & (dr == 0.0)
        dl = jnp.where(b0, 1.0, dl)
        dr = jnp.where(b0, 1.0, dr)
        return lft, r, dl, dr, dl + dr

    def group(g, carry):
        off = g * L
        vx = x_v[pl.ds(off, L)]
        vy = y_v[pl.ds(off, L)]
        va = a_v[pl.ds(off, L)]

        l0, r0, dl0, dr0, den0 = dim_interp(vx, H)
        l1, r1, dl1, dr1, den1 = dim_interp(vy, W)

        t = (va + math.pi) * (1.0 / OMEGA)
        it = jnp.clip(t.astype(jnp.int32), 0, A - 1)
        tick = plsc.load_gather(az_v, [it])
        theta = va - tick
        inv = 1.0 / (den0 * den1 * SIN_OMEGA)
        s1 = _sinpoly(OMEGA - theta) * inv
        s2 = _sinpoly(theta) * inv
        combos = (
            ((l0 * W + l1) * A, dr0 * dr1),
            ((l0 * W + r1) * A, dr0 * dl1),
            ((r0 * W + l1) * A, dl0 * dr1),
            ((r0 * W + r1) * A, dl0 * dl1),
        )
        w8 = []
        for k, (b, wc) in enumerate(combos):
            idx_v[pl.ds(k * L, L)] = b + it
            w8.append(wc * s1)
            w8.append(wc * s2)

        pltpu.async_copy(table.at[idx_v], rows_v, sem).wait()

        obase = iota * F + off * F
        for f in range(F):
            col = jnp.full((L,), f, jnp.int32)
            col2 = col + F
            acc = plsc.load_gather(rows_v, [row_ids[0], col]) * w8[0]
            acc = acc + plsc.load_gather(rows_v, [row_ids[0], col2]) * w8[1]
            for k in range(1, 4):
                acc = acc + plsc.load_gather(rows_v, [row_ids[k], col]) * w8[2 * k]
                acc = acc + plsc.load_gather(rows_v, [row_ids[k], col2]) * w8[2 * k + 1]
            plsc.store_scatter(out_v, [obase + f], acc)
        return carry

    lax.fori_loop(0, GROUPS, group, 0)
    pltpu.sync_copy(out_v, out_hbm.at[pl.ds(base * F, PTS_PER_W * F)])


def kernel(positions, angles, grid_values):
    x = positions[:, 0]
    y = positions[:, 1]
    pad = NP - N
    xs = jnp.pad(x, (0, pad))
    ys = jnp.pad(y, (0, pad))
    angs = jnp.pad(angles, (0, pad))
    t = jnp.transpose(grid_values, (2, 3, 1, 0)).reshape(H * W, A, F)
    table = jnp.concatenate([t, jnp.roll(t, -1, axis=1)], axis=-1)
    table = table.reshape(H * W * A, 2 * F)
    az = jnp.linspace(-math.pi, math.pi, A + 1)[:-1].astype(jnp.float32)
    az16 = jnp.pad(az, (0, 16 - A))
    out = _interp_sc(table, xs, ys, angs, az16)
    return out.reshape(NP, F)[:N]


# speedup vs baseline: 10.0101x; 1.0395x over previous
import functools
import math

import jax
import jax.numpy as jnp
from jax import lax
from jax.experimental import pallas as pl
from jax.experimental.pallas import tpu as pltpu
from jax.experimental.pallas import tpu_sc as plsc

N = 50000
F = 64
A = 8
H = 100
W = 100
NC = 2
NS = 16
NW = NC * NS
L = 16
GROUPS = 104
PTS_PER_W = GROUPS * L
NP = NW * PTS_PER_W
MACROS = GROUPS // 2
OMEGA = 2.0 * math.pi / A
SIN_OMEGA = math.sin(OMEGA)


def _sinpoly(t):
    t2 = t * t
    return t * (1.0 + t2 * (-1.0 / 6.0 + t2 * (1.0 / 120.0 - t2 * (1.0 / 5040.0))))


@functools.partial(
    pl.kernel,
    out_type=jax.ShapeDtypeStruct((NP * F,), jnp.float32),
    mesh=plsc.VectorSubcoreMesh(
        core_axis_name="c", subcore_axis_name="s", num_cores=NC, num_subcores=NS
    ),
    scratch_types=[
        pltpu.VMEM((PTS_PER_W,), jnp.float32),
        pltpu.VMEM((PTS_PER_W,), jnp.float32),
        pltpu.VMEM((PTS_PER_W,), jnp.float32),
        pltpu.VMEM((16,), jnp.float32),
        pltpu.VMEM((GROUPS * 4 * L,), jnp.int32),
        pltpu.VMEM((GROUPS * 8 * L,), jnp.float32),
        pltpu.VMEM((8 * L, 2 * F), jnp.float32),
        pltpu.VMEM((8 * L, 2 * F), jnp.float32),
        pltpu.VMEM((2 * L * F,), jnp.float32),
        pltpu.VMEM((2 * L * F,), jnp.float32),
        pltpu.SemaphoreType.DMA,
        pltpu.SemaphoreType.DMA,
    ],
    compiler_params=pltpu.CompilerParams(needs_layout_passes=False),
)
def _interp_sc(table, xs, ys, angs, az, out_hbm,
               x_v, y_v, a_v, az_v, idx_v, w_v, rows0_v, rows1_v,
               out0_v, out1_v, sem0, sem1):
    wid = lax.axis_index("s") * NC + lax.axis_index("c")
    base = pl.multiple_of(wid * PTS_PER_W, 8)
    pltpu.sync_copy(xs.at[pl.ds(base, PTS_PER_W)], x_v)
    pltpu.sync_copy(ys.at[pl.ds(base, PTS_PER_W)], y_v)
    pltpu.sync_copy(angs.at[pl.ds(base, PTS_PER_W)], a_v)
    pltpu.sync_copy(az, az_v)

    iota = lax.iota(jnp.int32, L)

    def dim_interp(v, n):
        cv = v.astype(jnp.int32)
        ceil = jnp.where(v > cv.astype(jnp.float32), cv + 1, cv)
        r = jnp.minimum(ceil, n - 1)
        lft = jnp.maximum(r - 1, 0)
        dl = jnp.maximum(v - lft.astype(jnp.float32), 0.0)
        dr = jnp.maximum(r.astype(jnp.float32) - v, 0.0)
        b0 = (dl == 0.0) & (dr == 0.0)
        dl = jnp.where(b0, 1.0, dl)
        dr = jnp.where(b0, 1.0, dr)
        return lft, r, dl, dr, dl + dr

    def phase_a(g, carry):
        off = g * L
        vx = x_v[pl.ds(off, L)]
        vy = y_v[pl.ds(off, L)]
        va = a_v[pl.ds(off, L)]

        l0, r0, dl0, dr0, den0 = dim_interp(vx, H)
        l1, r1, dl1, dr1, den1 = dim_interp(vy, W)

        t = (va + math.pi) * (1.0 / OMEGA)
        it = jnp.clip(t.astype(jnp.int32), 0, A - 1)
        tick = plsc.load_gather(az_v, [it])
        theta = va - tick
        inv = 1.0 / (den0 * den1 * SIN_OMEGA)
        s1 = _sinpoly(OMEGA - theta) * inv
        s2 = _sinpoly(theta) * inv

        combos = (
            ((l0 * W + l1) * A, dr0 * dr1),
            ((l0 * W + r1) * A, dr0 * dl1),
            ((r0 * W + l1) * A, dl0 * dr1),
            ((r0 * W + r1) * A, dl0 * dl1),
        )
        ibase = g * (4 * L)
        wbase = g * (8 * L)
        for k, (b, wc) in enumerate(combos):
            idx_v[pl.ds(ibase + k * L, L)] = b + it
            w_v[pl.ds(wbase + (2 * k) * L, L)] = wc * s1
            w_v[pl.ds(wbase + (2 * k + 1) * L, L)] = wc * s2
        return carry

    lax.fori_loop(0, GROUPS, phase_a, 0)

    row_ids = [iota + sub * 4 * L + k * L for sub in range(2) for k in range(4)]
    obase = [iota * F + sub * L * F for sub in range(2)]

    def fire(m, rows, sem):
        pltpu.async_copy(
            table.at[idx_v.at[pl.ds(m * (4 * 2 * L), 4 * 2 * L)]],
            rows, sem)

    def accum(m, rows, out):
        for sub in range(2):
            g2 = m * 2 + sub
            wb = g2 * (8 * L)
            w8 = [w_v[pl.ds(wb + j * L, L)] for j in range(8)]
            rid = row_ids[sub * 4:sub * 4 + 4]
            for f in range(F):
                col = jnp.full((L,), f, jnp.int32)
                col2 = col + F
                acc = plsc.load_gather(rows, [rid[0], col]) * w8[0]
                acc = acc + plsc.load_gather(rows, [rid[0], col2]) * w8[1]
                for k in range(1, 4):
                    acc = acc + plsc.load_gather(rows, [rid[k], col]) * w8[2 * k]
                    acc = acc + plsc.load_gather(rows, [rid[k], col2]) * w8[2 * k + 1]
                plsc.store_scatter(out, [obase[sub] + f], acc)
        pltpu.sync_copy(out,
                        out_hbm.at[pl.ds(base * F + m * (2 * L * F), 2 * L * F)])

    fire(0, rows0_v, sem0)

    def phase_b(i, carry):
        m0 = i * 2
        fire(m0 + 1, rows1_v, sem1)
        pltpu.make_async_copy(table.at[idx_v.at[pl.ds(0, 8 * L)]],
                              rows0_v, sem0).wait()
        accum(m0, rows0_v, out0_v)

        @pl.when(i < MACROS // 2 - 1)
        def _():
            fire(m0 + 2, rows0_v, sem0)

        pltpu.make_async_copy(table.at[idx_v.at[pl.ds(0, 8 * L)]],
                              rows1_v, sem1).wait()
        accum(m0 + 1, rows1_v, out1_v)
        return carry

    lax.fori_loop(0, MACROS // 2, phase_b, 0)


def kernel(positions, angles, grid_values):
    x = positions[:, 0]
    y = positions[:, 1]
    pad = NP - N
    xs = jnp.pad(x, (0, pad))
    ys = jnp.pad(y, (0, pad))
    angs = jnp.pad(angles, (0, pad))
    t = jnp.transpose(grid_values, (2, 3, 1, 0)).reshape(H * W, A, F)
    table = jnp.concatenate([t, jnp.roll(t, -1, axis=1)], axis=-1)
    table = table.reshape(H * W * A, 2 * F)
    az = jnp.linspace(-math.pi, math.pi, A + 1)[:-1].astype(jnp.float32)
    az16 = jnp.pad(az, (0, 16 - A))
    out = _interp_sc(table, xs, ys, angs, az16)
    return out.reshape(NP, F)[:N]


# speedup vs baseline: 10.7601x; 1.0749x over previous
import functools
import math

import jax
import jax.numpy as jnp
from jax import lax
from jax.experimental import pallas as pl
from jax.experimental.pallas import tpu as pltpu
from jax.experimental.pallas import tpu_sc as plsc

N = 50000
F = 64
A = 8
H = 100
W = 100
NC = 2
NS = 16
NW = NC * NS
L = 16
GROUPS = 104
PTS_PER_W = GROUPS * L
NP = NW * PTS_PER_W
MACROS = GROUPS // 2
OMEGA = 2.0 * math.pi / A
SIN_OMEGA = math.sin(OMEGA)


def _sinpoly(t):
    t2 = t * t
    return t * (1.0 + t2 * (-1.0 / 6.0 + t2 * (1.0 / 120.0 - t2 * (1.0 / 5040.0))))


@functools.partial(
    pl.kernel,
    out_type=jax.ShapeDtypeStruct((NP * F,), jnp.float32),
    mesh=plsc.VectorSubcoreMesh(
        core_axis_name="c", subcore_axis_name="s", num_cores=NC, num_subcores=NS
    ),
    scratch_types=[
        pltpu.VMEM((PTS_PER_W,), jnp.float32),
        pltpu.VMEM((PTS_PER_W,), jnp.float32),
        pltpu.VMEM((PTS_PER_W,), jnp.float32),
        pltpu.VMEM((16,), jnp.float32),
        pltpu.VMEM((GROUPS * 4 * L,), jnp.int32),
        pltpu.VMEM((GROUPS * 8 * L,), jnp.float32),
        pltpu.VMEM((8 * L, 2 * F), jnp.float32),
        pltpu.VMEM((8 * L, 2 * F), jnp.float32),
        pltpu.VMEM((2 * L * F,), jnp.float32),
        pltpu.VMEM((2 * L * F,), jnp.float32),
        pltpu.SemaphoreType.DMA,
        pltpu.SemaphoreType.DMA,
    ],
    compiler_params=pltpu.CompilerParams(needs_layout_passes=False),
)
def _interp_sc(table, xs, ys, angs, az, out_hbm,
               x_v, y_v, a_v, az_v, idx_v, w_v, rows0_v, rows1_v,
               out0_v, out1_v, sem0, sem1):
    wid = lax.axis_index("s") * NC + lax.axis_index("c")
    base = pl.multiple_of(wid * PTS_PER_W, 8)
    pltpu.sync_copy(xs.at[pl.ds(base, PTS_PER_W)], x_v)
    pltpu.sync_copy(ys.at[pl.ds(base, PTS_PER_W)], y_v)
    pltpu.sync_copy(angs.at[pl.ds(base, PTS_PER_W)], a_v)
    pltpu.sync_copy(az, az_v)

    iota = lax.iota(jnp.int32, L)

    def dim_interp(v, n):
        cv = v.astype(jnp.int32)
        ceil = jnp.where(v > cv.astype(jnp.float32), cv + 1, cv)
        r = jnp.minimum(ceil, n - 1)
        lft = jnp.maximum(r - 1, 0)
        dl = jnp.maximum(v - lft.astype(jnp.float32), 0.0)
        dr = jnp.maximum(r.astype(jnp.float32) - v, 0.0)
        b0 = (dl == 0.0) & (dr == 0.0)
        dl = jnp.where(b0, 1.0, dl)
        dr = jnp.where(b0, 1.0, dr)
        return lft, r, dl, dr, dl + dr

    def phase_a(g, carry):
        off = g * L
        vx = x_v[pl.ds(off, L)]
        vy = y_v[pl.ds(off, L)]
        va = a_v[pl.ds(off, L)]

        l0, r0, dl0, dr0, den0 = dim_interp(vx, H)
        l1, r1, dl1, dr1, den1 = dim_interp(vy, W)

        t = (va + math.pi) * (1.0 / OMEGA)
        it = jnp.clip(t.astype(jnp.int32), 0, A - 1)
        tick = plsc.load_gather(az_v, [it])
        theta = va - tick
        inv = 1.0 / (den0 * den1 * SIN_OMEGA)
        s1 = _sinpoly(OMEGA - theta) * inv
        s2 = _sinpoly(theta) * inv

        combos = (
            ((l0 * W + l1) * A, dr0 * dr1),
            ((l0 * W + r1) * A, dr0 * dl1),
            ((r0 * W + l1) * A, dl0 * dr1),
            ((r0 * W + r1) * A, dl0 * dl1),
        )
        ibase = g * (4 * L)
        wbase = g * (8 * L)
        for k, (b, wc) in enumerate(combos):
            idx_v[pl.ds(ibase + k * L, L)] = b + it
            w_v[pl.ds(wbase + (2 * k) * L, L)] = wc * s1
            w_v[pl.ds(wbase + (2 * k + 1) * L, L)] = wc * s2
        return carry

    lax.fori_loop(0, GROUPS, phase_a, 0)

    splat_ids = [jnp.full((L, 1), p, jnp.int32) for p in range(L)]
    _splat_dnums = lax.GatherDimensionNumbers(
        offset_dims=(), collapsed_slice_dims=(0,), start_index_map=(0,))

    def _splat(vec, pid):
        return lax.gather(vec, pid, _splat_dnums, (1,),
                          mode=lax.GatherScatterMode.PROMISE_IN_BOUNDS)

    NCH = F // L

    def fire(m, rows, sem):
        pltpu.async_copy(
            table.at[idx_v.at[pl.ds(m * (4 * 2 * L), 4 * 2 * L)]],
            rows, sem)

    def accum(m, rows, out):
        for sub in range(2):
            g2 = m * 2 + sub
            wb = g2 * (8 * L)
            w8 = [w_v[pl.ds(wb + j * L, L)] for j in range(8)]
            for p in range(L):
                ws = [_splat(w, splat_ids[p]) for w in w8]
                rid = [sub * 4 * L + k * L + p for k in range(4)]
                ob = (sub * L + p) * F
                for c in range(NCH):
                    acc = None
                    for k in range(4):
                        r1 = rows[rid[k], pl.ds(c * L, L)] * ws[2 * k]
                        r2 = rows[rid[k], pl.ds(F + c * L, L)] * ws[2 * k + 1]
                        acc = r1 + r2 if acc is None else acc + (r1 + r2)
                    out[pl.ds(ob + c * L, L)] = acc
        pltpu.sync_copy(out,
                        out_hbm.at[pl.ds(base * F + m * (2 * L * F), 2 * L * F)])

    fire(0, rows0_v, sem0)

    def phase_b(i, carry):
        m0 = i * 2
        fire(m0 + 1, rows1_v, sem1)
        pltpu.make_async_copy(table.at[idx_v.at[pl.ds(0, 8 * L)]],
                              rows0_v, sem0).wait()
        accum(m0, rows0_v, out0_v)

        @pl.when(i < MACROS // 2 - 1)
        def _():
            fire(m0 + 2, rows0_v, sem0)

        pltpu.make_async_copy(table.at[idx_v.at[pl.ds(0, 8 * L)]],
                              rows1_v, sem1).wait()
        accum(m0 + 1, rows1_v, out1_v)
        return carry

    lax.fori_loop(0, MACROS // 2, phase_b, 0)


def kernel(positions, angles, grid_values):
    x = positions[:, 0]
    y = positions[:, 1]
    pad = NP - N
    xs = jnp.pad(x, (0, pad))
    ys = jnp.pad(y, (0, pad))
    angs = jnp.pad(angles, (0, pad))
    t = jnp.transpose(grid_values, (2, 3, 1, 0)).reshape(H * W, A, F)
    table = jnp.concatenate([t, jnp.roll(t, -1, axis=1)], axis=-1)
    table = table.reshape(H * W * A, 2 * F)
    az = jnp.linspace(-math.pi, math.pi, A + 1)[:-1].astype(jnp.float32)
    az16 = jnp.pad(az, (0, 16 - A))
    out = _interp_sc(table, xs, ys, angs, az16)
    return out.reshape(NP, F)[:N]
